# baseline (speedup 1.0000x reference)
import functools

import jax
import jax.numpy as jnp
from jax import lax
from jax.experimental import pallas as pl
from jax.experimental.pallas import tpu as pltpu
from jax.experimental.pallas import tpu_sc as plsc

N_USER = 50000
N_ITEM = 50000
N = N_USER + N_ITEM
N_PAD = N
D = 16
E = 3200000
NC, NS = 2, 16
NW = NC * NS
CJ, CL = 4, 128
CHUNK = CJ * CL
NCHUNK_PAD = 6336
E_PAD = NCHUNK_PAD * CHUNK
TASKS = NCHUNK_PAD // NW
RING = 3
T3 = TASKS // RING
RPS = N_PAD // NS
WB = 250


def _layer_call(h, src3, dst3, w3):
    mesh = plsc.VectorSubcoreMesh(core_axis_name="c", subcore_axis_name="s")

    @functools.partial(
        pl.kernel,
        out_type=jax.ShapeDtypeStruct((NC, N_PAD, D), jnp.float32),
        mesh=mesh,
        compiler_params=pltpu.CompilerParams(use_tc_tiling_on_sc=False),
        scratch_types=[
            pltpu.VMEM((RING, CJ, CL), jnp.int32),
            pltpu.VMEM((RING, CJ, CL), jnp.int32),
            pltpu.VMEM((RING, CHUNK), jnp.float32),
            pltpu.VMEM((RING, CHUNK, D), jnp.float32),
            pltpu.VMEM_SHARED((N_PAD, D), jnp.float32),
            pltpu.SemaphoreType.DMA,
            pltpu.SemaphoreType.DMA,
            pltpu.SemaphoreType.DMA,
            pltpu.SemaphoreType.DMA,
            pltpu.SemaphoreType.DMA,
            pltpu.SemaphoreType.DMA,
        ],
    )
    def k(h_hbm, src_hbm, dst_hbm, w_hbm, out_hbm,
          src_v, dst_v, w_v, rows_v, acc,
          gsem0, gsem1, gsem2, ssem0, ssem1, ssem2):
        cid = lax.axis_index("c")
        sid = lax.axis_index("s")
        wid = sid * NC + cid
        gsems = [gsem0, gsem1, gsem2]
        ssems = [ssem0, ssem1, ssem2]

        def load_idx(slot, chunk):
            pltpu.sync_copy(src_hbm.at[chunk], src_v.at[slot])
            pltpu.sync_copy(w_hbm.at[chunk], w_v.at[slot])
            pltpu.sync_copy(dst_hbm.at[chunk], dst_v.at[slot])

        def issue_gathers(slot):
            return [pltpu.async_copy(h_hbm.at[src_v.at[slot, j]],
                                     rows_v.at[slot, pl.ds(j * CL, CL)],
                                     gsems[slot])
                    for j in range(CJ)]

        def issue_scatters(slot):
            return [pltpu.async_copy(rows_v.at[slot, pl.ds(j * CL, CL)],
                                     acc.at[dst_v.at[slot, j]],
                                     ssems[slot], add=True)
                    for j in range(CJ)]

        def zero_body(i, _):
            rows_v[0, i] = jnp.zeros((D,), jnp.float32)
            return 0
        lax.fori_loop(0, WB, zero_body, 0)
        for r in range(RPS // WB):
            pltpu.sync_copy(rows_v.at[0, pl.ds(0, WB)],
                            acc.at[pl.ds(sid * RPS + r * WB, WB)])

        plsc.subcore_barrier()

        def mul_rows(slot):
            def mul_g(g, _):
                w16 = w_v[slot, pl.ds(g * 16, 16)]
                base = g * 16
                for e in range(16):
                    wsp = lax.gather(
                        w16, jnp.full((16, 1), e, jnp.int32),
                        lax.GatherDimensionNumbers(
                            offset_dims=(), collapsed_slice_dims=(0,),
                            start_index_map=(0,)),
                        (1,), mode=lax.GatherScatterMode.PROMISE_IN_BOUNDS)
                    rows_v[slot, base + e] = rows_v[slot, base + e] * wsp
                return 0
            lax.fori_loop(0, CHUNK // 16, mul_g, 0)

        def body(g, _):
            base = wid + g * RING * NW
            gds = []
            for slot in range(RING):
                load_idx(slot, base + slot * NW)
                gds.append(issue_gathers(slot))
            sds = []
            for slot in range(RING):
                for d_ in gds[slot]:
                    d_.wait()
                mul_rows(slot)
                sds.append(issue_scatters(slot))
            for lst in sds:
                for d_ in lst:
                    d_.wait()
            return 0
        lax.fori_loop(0, T3, body, 0)
        plsc.subcore_barrier()

        for r in range(RPS // WB):
            base = sid * RPS + r * WB
            pltpu.sync_copy(acc.at[pl.ds(base, WB)], rows_v.at[0, pl.ds(0, WB)])
            pltpu.sync_copy(rows_v.at[0, pl.ds(0, WB)],
                            out_hbm.at[cid, pl.ds(base, WB)])

    return k(h, src3, dst3, w3)


def _combine_call(parts2, agg2):
    R, C = agg2.shape

    def ck(p_ref, a_ref, h_ref, g_ref):
        hh = p_ref[0] + p_ref[1]
        h_ref[...] = hh
        g_ref[...] = a_ref[...] + hh

    return pl.pallas_call(
        ck,
        out_shape=[jax.ShapeDtypeStruct((R, C), jnp.float32),
                   jax.ShapeDtypeStruct((R, C), jnp.float32)],
    )(parts2, agg2)


def _batch_gather_call(agg, uid2, iid2, nid2):
    mesh = plsc.VectorSubcoreMesh(core_axis_name="c", subcore_axis_name="s")

    @functools.partial(
        pl.kernel,
        out_type=[jax.ShapeDtypeStruct((NW, 128, D), jnp.float32)] * 3,
        mesh=mesh,
        compiler_params=pltpu.CompilerParams(use_tc_tiling_on_sc=False),
        scratch_types=[
            pltpu.VMEM((128,), jnp.int32),
            pltpu.VMEM((128, D), jnp.float32),
            pltpu.SemaphoreType.DMA,
        ],
    )
    def k(agg_hbm, u_hbm, i_hbm, n_hbm, bu_out, bi_out, bn_out,
          idx_v, rows_v, sem):
        cid = lax.axis_index("c")
        sid = lax.axis_index("s")
        wid = sid * NC + cid
        for ids_hbm, out_hbm in ((u_hbm, bu_out), (i_hbm, bi_out),
                                 (n_hbm, bn_out)):
            pltpu.sync_copy(ids_hbm.at[wid], idx_v)
            pltpu.async_copy(agg_hbm.at[idx_v], rows_v, sem).wait()
            pltpu.sync_copy(rows_v, out_hbm.at[wid])

    return k(agg, uid2, iid2, nid2)


def _loss_call(bu, bi, bn):
    def lk(bu_ref, bi_ref, bn_ref, o_ref):
        z = jnp.sum(bu_ref[...] * (bi_ref[...] - bn_ref[...]), axis=1)
        z = z * (1.0 / 16.0)
        sp = jnp.maximum(-z, 0.0) + jnp.log1p(jnp.exp(-jnp.abs(z)))
        o_ref[...] = jnp.mean(sp).reshape(1, 1)

    return pl.pallas_call(
        lk, out_shape=jax.ShapeDtypeStruct((1, 1), jnp.float32))(bu, bi, bn)


def kernel(user_embeddings, item_embeddings, edge_weight, edge_index,
           user_ids, item_ids, neg_item_ids):
    x = jnp.concatenate([user_embeddings, item_embeddings], axis=0)
    pad = E_PAD - E
    src = jnp.concatenate([edge_index[0], jnp.zeros((pad,), jnp.int32)])
    dst = jnp.concatenate([edge_index[1], jnp.zeros((pad,), jnp.int32)])
    w = jnp.concatenate([edge_weight, jnp.zeros((pad,), jnp.float32)])
    src3 = src.reshape(NCHUNK_PAD, CJ, CL)
    dst3 = dst.reshape(NCHUNK_PAD, CJ, CL)
    w3 = w.reshape(NCHUNK_PAD, CHUNK)

    h = x
    agg = x.reshape(N_PAD * D // 128, 128)
    for _ in range(3):
        parts = _layer_call(h, src3, dst3, w3)
        h2, agg = _combine_call(parts.reshape(NC, N_PAD * D // 128, 128), agg)
        h = h2.reshape(N_PAD, D)

    uid2 = user_ids.reshape(NW, 128)
    iid2 = (item_ids + N_USER).reshape(NW, 128)
    nid2 = (neg_item_ids + N_USER).reshape(NW, 128)
    bu, bi, bn = _batch_gather_call(agg.reshape(N_PAD, D), uid2, iid2, nid2)
    loss2 = _loss_call(bu.reshape(4096, D), bi.reshape(4096, D),
                       bn.reshape(4096, D))
    return loss2[0, 0]


# speedup vs baseline: 1.0227x; 1.0227x over previous
import functools

import jax
import jax.numpy as jnp
from jax import lax
from jax.experimental import pallas as pl
from jax.experimental.pallas import tpu as pltpu
from jax.experimental.pallas import tpu_sc as plsc

N_USER = 50000
N_ITEM = 50000
N = N_USER + N_ITEM
N_PAD = N
D = 16
E = 3200000
NC, NS = 2, 16
NW = NC * NS
CJ, CL = 12, 128
CHUNK = CJ * CL
NCHUNK_PAD = 2112
E_PAD = NCHUNK_PAD * CHUNK
TASKS = NCHUNK_PAD // NW
RPS = N_PAD // NS
WB = 625


def _layer_call(h, meta3, w3):
    mesh = plsc.VectorSubcoreMesh(core_axis_name="c", subcore_axis_name="s")

    @functools.partial(
        pl.kernel,
        out_type=jax.ShapeDtypeStruct((NC, N_PAD, D), jnp.float32),
        mesh=mesh,
        compiler_params=pltpu.CompilerParams(use_tc_tiling_on_sc=False),
        scratch_types=[
            pltpu.VMEM((2 * CJ, CL), jnp.int32),
            pltpu.VMEM((CHUNK,), jnp.float32),
            pltpu.VMEM((CHUNK, D), jnp.float32),
            pltpu.VMEM_SHARED((N_PAD, D), jnp.float32),
            pltpu.SemaphoreType.DMA,
            pltpu.SemaphoreType.DMA,
        ],
    )
    def k(h_hbm, meta_hbm, w_hbm, out_hbm, meta_v, w_v, rows_v, acc, gsem, ssem):
        cid = lax.axis_index("c")
        sid = lax.axis_index("s")
        wid = sid * NC + cid

        def zero_body(i, _):
            rows_v[i] = jnp.zeros((D,), jnp.float32)
            return 0
        lax.fori_loop(0, WB, zero_body, 0)
        for r in range(RPS // WB):
            pltpu.sync_copy(rows_v.at[pl.ds(0, WB)],
                            acc.at[pl.ds(sid * RPS + r * WB, WB)])
        plsc.subcore_barrier()

        def body(t, _):
            chunk = wid + t * NW
            pltpu.sync_copy(meta_hbm.at[chunk], meta_v)
            pltpu.sync_copy(w_hbm.at[chunk], w_v)
            descs = [pltpu.async_copy(h_hbm.at[meta_v.at[j]],
                                      rows_v.at[pl.ds(j * CL, CL)], gsem)
                     for j in range(CJ)]
            for d_ in descs:
                d_.wait()

            def mul_g(g, _):
                w16 = w_v[pl.ds(g * 16, 16)]
                base = g * 16
                for e in range(16):
                    wsp = lax.gather(
                        w16, jnp.full((16, 1), e, jnp.int32),
                        lax.GatherDimensionNumbers(
                            offset_dims=(), collapsed_slice_dims=(0,),
                            start_index_map=(0,)),
                        (1,), mode=lax.GatherScatterMode.PROMISE_IN_BOUNDS)
                    rows_v[base + e] = rows_v[base + e] * wsp
                return 0
            lax.fori_loop(0, CHUNK // 16, mul_g, 0)

            sdescs = [pltpu.async_copy(rows_v.at[pl.ds(j * CL, CL)],
                                       acc.at[meta_v.at[CJ + j]], ssem,
                                       add=True)
                      for j in range(CJ)]
            for d_ in sdescs:
                d_.wait()
            return 0
        lax.fori_loop(0, TASKS, body, 0)
        plsc.subcore_barrier()

        for r in range(RPS // WB):
            base = sid * RPS + r * WB
            pltpu.sync_copy(acc.at[pl.ds(base, WB)], rows_v.at[pl.ds(0, WB)])
            pltpu.sync_copy(rows_v.at[pl.ds(0, WB)],
                            out_hbm.at[cid, pl.ds(base, WB)])

    return k(h, meta3, w3)


def _combine_call(parts2, agg2):
    R, C = agg2.shape

    def ck(p_ref, a_ref, h_ref, g_ref):
        hh = p_ref[0] + p_ref[1]
        h_ref[...] = hh
        g_ref[...] = a_ref[...] + hh

    return pl.pallas_call(
        ck,
        out_shape=[jax.ShapeDtypeStruct((R, C), jnp.float32),
                   jax.ShapeDtypeStruct((R, C), jnp.float32)],
    )(parts2, agg2)


def _batch_gather_call(agg, uid2, iid2, nid2):
    mesh = plsc.VectorSubcoreMesh(core_axis_name="c", subcore_axis_name="s")

    @functools.partial(
        pl.kernel,
        out_type=[jax.ShapeDtypeStruct((NW, 128, D), jnp.float32)] * 3,
        mesh=mesh,
        compiler_params=pltpu.CompilerParams(use_tc_tiling_on_sc=False),
        scratch_types=[
            pltpu.VMEM((128,), jnp.int32),
            pltpu.VMEM((128, D), jnp.float32),
            pltpu.SemaphoreType.DMA,
        ],
    )
    def k(agg_hbm, u_hbm, i_hbm, n_hbm, bu_out, bi_out, bn_out,
          idx_v, rows_v, sem):
        cid = lax.axis_index("c")
        sid = lax.axis_index("s")
        wid = sid * NC + cid
        for ids_hbm, out_hbm in ((u_hbm, bu_out), (i_hbm, bi_out),
                                 (n_hbm, bn_out)):
            pltpu.sync_copy(ids_hbm.at[wid], idx_v)
            pltpu.async_copy(agg_hbm.at[idx_v], rows_v, sem).wait()
            pltpu.sync_copy(rows_v, out_hbm.at[wid])

    return k(agg, uid2, iid2, nid2)


def _loss_call(bu, bi, bn):
    def lk(bu_ref, bi_ref, bn_ref, o_ref):
        z = jnp.sum(bu_ref[...] * (bi_ref[...] - bn_ref[...]), axis=1)
        z = z * (1.0 / 16.0)
        sp = jnp.maximum(-z, 0.0) + jnp.log1p(jnp.exp(-jnp.abs(z)))
        o_ref[...] = jnp.mean(sp).reshape(1, 1)

    return pl.pallas_call(
        lk, out_shape=jax.ShapeDtypeStruct((1, 1), jnp.float32))(bu, bi, bn)


def kernel(user_embeddings, item_embeddings, edge_weight, edge_index,
           user_ids, item_ids, neg_item_ids):
    x = jnp.concatenate([user_embeddings, item_embeddings], axis=0)
    pad = E_PAD - E
    src = jnp.concatenate([edge_index[0], jnp.zeros((pad,), jnp.int32)])
    dst = jnp.concatenate([edge_index[1], jnp.zeros((pad,), jnp.int32)])
    w = jnp.concatenate([edge_weight, jnp.zeros((pad,), jnp.float32)])
    w3 = w.reshape(NCHUNK_PAD, CHUNK)
    meta3 = jnp.concatenate([src.reshape(NCHUNK_PAD, CJ, CL),
                             dst.reshape(NCHUNK_PAD, CJ, CL)], axis=1)

    h = x
    agg = x.reshape(N_PAD * D // 128, 128)
    for _ in range(3):
        parts = _layer_call(h, meta3, w3)
        h2, agg = _combine_call(parts.reshape(NC, N_PAD * D // 128, 128), agg)
        h = h2.reshape(N_PAD, D)

    uid2 = user_ids.reshape(NW, 128)
    iid2 = (item_ids + N_USER).reshape(NW, 128)
    nid2 = (neg_item_ids + N_USER).reshape(NW, 128)
    bu, bi, bn = _batch_gather_call(agg.reshape(N_PAD, D), uid2, iid2, nid2)
    loss2 = _loss_call(bu.reshape(4096, D), bi.reshape(4096, D),
                       bn.reshape(4096, D))
    return loss2[0, 0]


# speedup vs baseline: 1.5165x; 1.4828x over previous
import functools

import jax
import jax.numpy as jnp
from jax import lax
from jax.experimental import pallas as pl
from jax.experimental.pallas import tpu as pltpu
from jax.experimental.pallas import tpu_sc as plsc

N_USER = 50000
N_ITEM = 50000
N = N_USER + N_ITEM
N_PAD = N
D = 16
E = 3200000
NC, NS = 2, 16
NW = NC * NS
CJ, CL = 8, 128
CHUNK = CJ * CL
NCHUNK_PAD = 3136
NCHUNK_META = 3168
E_PAD = NCHUNK_PAD * CHUNK
TASKS = NCHUNK_PAD // NW
RPS = N_PAD // NS
WB = 625


def _layer_call(h, meta3, w3):
    mesh = plsc.VectorSubcoreMesh(core_axis_name="c", subcore_axis_name="s")

    @functools.partial(
        pl.kernel,
        out_type=jax.ShapeDtypeStruct((NC, N_PAD, D), jnp.float32),
        mesh=mesh,
        compiler_params=pltpu.CompilerParams(use_tc_tiling_on_sc=False),
        scratch_types=[
            pltpu.VMEM((2, 2 * CJ, CL), jnp.int32),
            pltpu.VMEM((2, CHUNK), jnp.float32),
            pltpu.VMEM((CHUNK, D), jnp.float32),
            pltpu.VMEM_SHARED((N_PAD, D), jnp.float32),
            pltpu.SemaphoreType.DMA,
            pltpu.SemaphoreType.DMA,
            pltpu.SemaphoreType.DMA,
            pltpu.SemaphoreType.DMA,
        ],
    )
    def k(h_hbm, meta_hbm, w_hbm, out_hbm, meta_v, w_v, rows_v, acc,
          gsem, ssem, msem0, msem1):
        cid = lax.axis_index("c")
        sid = lax.axis_index("s")
        wid = sid * NC + cid

        def zero_body(i, _):
            rows_v[i] = jnp.zeros((D,), jnp.float32)
            return 0
        lax.fori_loop(0, WB, zero_body, 0)
        for r in range(RPS // WB):
            pltpu.sync_copy(rows_v.at[pl.ds(0, WB)],
                            acc.at[pl.ds(sid * RPS + r * WB, WB)])
        plsc.subcore_barrier()

        def issue_meta(slot, chunk, sem):
            return (pltpu.async_copy(meta_hbm.at[chunk], meta_v.at[slot], sem),
                    pltpu.async_copy(w_hbm.at[chunk], w_v.at[slot], sem))

        def drain_meta(slot, chunk, sem):
            pltpu.make_async_copy(meta_hbm.at[chunk], meta_v.at[slot],
                                  sem).wait()
            pltpu.make_async_copy(w_hbm.at[chunk], w_v.at[slot], sem).wait()

        def process(slot, chunk):
            descs = [pltpu.async_copy(h_hbm.at[meta_v.at[slot, j]],
                                      rows_v.at[pl.ds(j * CL, CL)], gsem)
                     for j in range(CJ)]
            return descs

        def mul_scatter(slot):
            def mul_g(g, _):
                w16 = w_v[slot, pl.ds(g * 16, 16)]
                base = g * 16
                for e in range(16):
                    wsp = lax.gather(
                        w16, jnp.full((16, 1), e, jnp.int32),
                        lax.GatherDimensionNumbers(
                            offset_dims=(), collapsed_slice_dims=(0,),
                            start_index_map=(0,)),
                        (1,), mode=lax.GatherScatterMode.PROMISE_IN_BOUNDS)
                    rows_v[base + e] = rows_v[base + e] * wsp
                return 0
            lax.fori_loop(0, CHUNK // 16, mul_g, 0)
            return [pltpu.async_copy(rows_v.at[pl.ds(j * CL, CL)],
                                     acc.at[meta_v.at[slot, CJ + j]], ssem,
                                     add=True)
                    for j in range(CJ)]

        issue_meta(0, wid, msem0)
        def body(p, _):
            ca = wid + (2 * p) * NW
            cb = ca + NW
            drain_meta(0, ca, msem0)
            gds = process(0, ca)
            mb = issue_meta(1, cb, msem1)
            for d_ in gds:
                d_.wait()
            sds = mul_scatter(0)
            for d_ in mb:
                d_.wait()
            for d_ in sds:
                d_.wait()
            gds = process(1, cb)
            issue_meta(0, ca + 2 * NW, msem0)
            for d_ in gds:
                d_.wait()
            sds = mul_scatter(1)
            for d_ in sds:
                d_.wait()
            return 0
        lax.fori_loop(0, TASKS // 2, body, 0)
        drain_meta(0, wid + TASKS * NW, msem0)
        plsc.subcore_barrier()

        for r in range(RPS // WB):
            base = sid * RPS + r * WB
            pltpu.sync_copy(acc.at[pl.ds(base, WB)], rows_v.at[pl.ds(0, WB)])
            pltpu.sync_copy(rows_v.at[pl.ds(0, WB)],
                            out_hbm.at[cid, pl.ds(base, WB)])

    return k(h, meta3, w3)


def _combine_call(parts2, agg2):
    R, C = agg2.shape

    def ck(p_ref, a_ref, h_ref, g_ref):
        hh = p_ref[0] + p_ref[1]
        h_ref[...] = hh
        g_ref[...] = a_ref[...] + hh

    return pl.pallas_call(
        ck,
        out_shape=[jax.ShapeDtypeStruct((R, C), jnp.float32),
                   jax.ShapeDtypeStruct((R, C), jnp.float32)],
    )(parts2, agg2)


def _batch_gather_call(agg, uid2, iid2, nid2):
    mesh = plsc.VectorSubcoreMesh(core_axis_name="c", subcore_axis_name="s")

    @functools.partial(
        pl.kernel,
        out_type=[jax.ShapeDtypeStruct((NW, 128, D), jnp.float32)] * 3,
        mesh=mesh,
        compiler_params=pltpu.CompilerParams(use_tc_tiling_on_sc=False),
        scratch_types=[
            pltpu.VMEM((128,), jnp.int32),
            pltpu.VMEM((128, D), jnp.float32),
            pltpu.SemaphoreType.DMA,
        ],
    )
    def k(agg_hbm, u_hbm, i_hbm, n_hbm, bu_out, bi_out, bn_out,
          idx_v, rows_v, sem):
        cid = lax.axis_index("c")
        sid = lax.axis_index("s")
        wid = sid * NC + cid
        for ids_hbm, out_hbm in ((u_hbm, bu_out), (i_hbm, bi_out),
                                 (n_hbm, bn_out)):
            pltpu.sync_copy(ids_hbm.at[wid], idx_v)
            pltpu.async_copy(agg_hbm.at[idx_v], rows_v, sem).wait()
            pltpu.sync_copy(rows_v, out_hbm.at[wid])

    return k(agg, uid2, iid2, nid2)


def _loss_call(bu, bi, bn):
    def lk(bu_ref, bi_ref, bn_ref, o_ref):
        z = jnp.sum(bu_ref[...] * (bi_ref[...] - bn_ref[...]), axis=1)
        z = z * (1.0 / 16.0)
        sp = jnp.maximum(-z, 0.0) + jnp.log1p(jnp.exp(-jnp.abs(z)))
        o_ref[...] = jnp.mean(sp).reshape(1, 1)

    return pl.pallas_call(
        lk, out_shape=jax.ShapeDtypeStruct((1, 1), jnp.float32))(bu, bi, bn)


def kernel(user_embeddings, item_embeddings, edge_weight, edge_index,
           user_ids, item_ids, neg_item_ids):
    x = jnp.concatenate([user_embeddings, item_embeddings], axis=0)
    pad = E_PAD - E
    src = jnp.concatenate([edge_index[0], jnp.zeros((pad,), jnp.int32)])
    dst = jnp.concatenate([edge_index[1], jnp.zeros((pad,), jnp.int32)])
    padm = NCHUNK_META * CHUNK - E
    srcm = jnp.concatenate([edge_index[0], jnp.zeros((padm,), jnp.int32)])
    dstm = jnp.concatenate([edge_index[1], jnp.zeros((padm,), jnp.int32)])
    w = jnp.concatenate([edge_weight, jnp.zeros((padm,), jnp.float32)])
    w3 = w.reshape(NCHUNK_META, CHUNK)
    meta3 = jnp.concatenate([srcm.reshape(NCHUNK_META, CJ, CL),
                             dstm.reshape(NCHUNK_META, CJ, CL)], axis=1)

    h = x
    agg = x.reshape(N_PAD * D // 128, 128)
    for _ in range(3):
        parts = _layer_call(h, meta3, w3)
        h2, agg = _combine_call(parts.reshape(NC, N_PAD * D // 128, 128), agg)
        h = h2.reshape(N_PAD, D)

    uid2 = user_ids.reshape(NW, 128)
    iid2 = (item_ids + N_USER).reshape(NW, 128)
    nid2 = (neg_item_ids + N_USER).reshape(NW, 128)
    bu, bi, bn = _batch_gather_call(agg.reshape(N_PAD, D), uid2, iid2, nid2)
    loss2 = _loss_call(bu.reshape(4096, D), bi.reshape(4096, D),
                       bn.reshape(4096, D))
    return loss2[0, 0]


# speedup vs baseline: 1.9295x; 1.2723x over previous
import functools

import jax
import jax.numpy as jnp
from jax import lax
from jax.experimental import pallas as pl
from jax.experimental.pallas import tpu as pltpu
from jax.experimental.pallas import tpu_sc as plsc

N_USER = 50000
N_ITEM = 50000
N = N_USER + N_ITEM
N_PAD = N
D = 16
E = 3200000
NC, NS = 2, 16
NW = NC * NS
CJ, CL = 8, 128
CHUNK = CJ * CL
NCHUNK_PAD = 3136
NCHUNK_META = 3168
E_PAD = NCHUNK_PAD * CHUNK
TASKS = NCHUNK_PAD // NW
RPS = N_PAD // NS
WB = 625


def _layer_call(h, meta3, w3):
    mesh = plsc.VectorSubcoreMesh(core_axis_name="c", subcore_axis_name="s")

    @functools.partial(
        pl.kernel,
        out_type=jax.ShapeDtypeStruct((NC, N_PAD, D), jnp.float32),
        mesh=mesh,
        compiler_params=pltpu.CompilerParams(use_tc_tiling_on_sc=False),
        scratch_types=[
            pltpu.VMEM((2, 2 * CJ, CL), jnp.int32),
            pltpu.VMEM((2, CHUNK), jnp.float32),
            pltpu.VMEM((CHUNK, D), jnp.float32),
            pltpu.VMEM_SHARED((N_PAD, D), jnp.float32),
            pltpu.SemaphoreType.DMA,
            pltpu.SemaphoreType.DMA,
            pltpu.SemaphoreType.DMA,
            pltpu.SemaphoreType.DMA,
        ],
    )
    def k(h_hbm, meta_hbm, w_hbm, out_hbm, meta_v, w_v, rows_v, acc,
          gsem, ssem, msem0, msem1):
        cid = lax.axis_index("c")
        sid = lax.axis_index("s")
        wid = sid * NC + cid

        def zero_body(i, _):
            rows_v[i] = jnp.zeros((D,), jnp.float32)
            return 0
        lax.fori_loop(0, WB, zero_body, 0)
        for r in range(RPS // WB):
            pltpu.sync_copy(rows_v.at[pl.ds(0, WB)],
                            acc.at[pl.ds(sid * RPS + r * WB, WB)])
        plsc.subcore_barrier()

        def issue_meta(slot, chunk, sem):
            return (pltpu.async_copy(meta_hbm.at[chunk], meta_v.at[slot], sem),
                    pltpu.async_copy(w_hbm.at[chunk], w_v.at[slot], sem))

        def drain_meta(slot, chunk, sem):
            pltpu.make_async_copy(meta_hbm.at[chunk], meta_v.at[slot],
                                  sem).wait()
            pltpu.make_async_copy(w_hbm.at[chunk], w_v.at[slot], sem).wait()

        def process(slot, chunk):
            descs = [pltpu.async_copy(h_hbm.at[meta_v.at[slot, j]],
                                      rows_v.at[pl.ds(j * CL, CL)], gsem)
                     for j in range(CJ)]
            return descs

        def mul_scatter(slot, gds):
            sds = []
            for j in range(CJ):
                gds[j].wait()

                def mul_g(g, _, j=j):
                    w16 = w_v[slot, pl.ds(j * CL + g * 16, 16)]
                    base = j * CL + g * 16
                    for e in range(16):
                        wsp = lax.gather(
                            w16, jnp.full((16, 1), e, jnp.int32),
                            lax.GatherDimensionNumbers(
                                offset_dims=(), collapsed_slice_dims=(0,),
                                start_index_map=(0,)),
                            (1,),
                            mode=lax.GatherScatterMode.PROMISE_IN_BOUNDS)
                        rows_v[base + e] = rows_v[base + e] * wsp
                    return 0
                lax.fori_loop(0, CL // 16, mul_g, 0)
                sds.append(pltpu.async_copy(rows_v.at[pl.ds(j * CL, CL)],
                                            acc.at[meta_v.at[slot, CJ + j]],
                                            ssem, add=True))
            return sds

        issue_meta(0, wid, msem0)
        def body(p, _):
            ca = wid + (2 * p) * NW
            cb = ca + NW
            drain_meta(0, ca, msem0)
            gds = process(0, ca)
            mb = issue_meta(1, cb, msem1)
            sds = mul_scatter(0, gds)
            for d_ in mb:
                d_.wait()
            for d_ in sds:
                d_.wait()
            gds = process(1, cb)
            issue_meta(0, ca + 2 * NW, msem0)
            sds = mul_scatter(1, gds)
            for d_ in sds:
                d_.wait()
            return 0
        lax.fori_loop(0, TASKS // 2, body, 0)
        drain_meta(0, wid + TASKS * NW, msem0)
        plsc.subcore_barrier()

        for r in range(RPS // WB):
            base = sid * RPS + r * WB
            pltpu.sync_copy(acc.at[pl.ds(base, WB)], rows_v.at[pl.ds(0, WB)])
            pltpu.sync_copy(rows_v.at[pl.ds(0, WB)],
                            out_hbm.at[cid, pl.ds(base, WB)])

    return k(h, meta3, w3)


def _combine_call(parts2, agg2):
    R, C = agg2.shape

    def ck(p_ref, a_ref, h_ref, g_ref):
        hh = p_ref[0] + p_ref[1]
        h_ref[...] = hh
        g_ref[...] = a_ref[...] + hh

    return pl.pallas_call(
        ck,
        out_shape=[jax.ShapeDtypeStruct((R, C), jnp.float32),
                   jax.ShapeDtypeStruct((R, C), jnp.float32)],
    )(parts2, agg2)


def _batch_gather_call(agg, uid2, iid2, nid2):
    mesh = plsc.VectorSubcoreMesh(core_axis_name="c", subcore_axis_name="s")

    @functools.partial(
        pl.kernel,
        out_type=[jax.ShapeDtypeStruct((NW, 128, D), jnp.float32)] * 3,
        mesh=mesh,
        compiler_params=pltpu.CompilerParams(use_tc_tiling_on_sc=False),
        scratch_types=[
            pltpu.VMEM((128,), jnp.int32),
            pltpu.VMEM((128, D), jnp.float32),
            pltpu.SemaphoreType.DMA,
        ],
    )
    def k(agg_hbm, u_hbm, i_hbm, n_hbm, bu_out, bi_out, bn_out,
          idx_v, rows_v, sem):
        cid = lax.axis_index("c")
        sid = lax.axis_index("s")
        wid = sid * NC + cid
        for ids_hbm, out_hbm in ((u_hbm, bu_out), (i_hbm, bi_out),
                                 (n_hbm, bn_out)):
            pltpu.sync_copy(ids_hbm.at[wid], idx_v)
            pltpu.async_copy(agg_hbm.at[idx_v], rows_v, sem).wait()
            pltpu.sync_copy(rows_v, out_hbm.at[wid])

    return k(agg, uid2, iid2, nid2)


def _loss_call(bu, bi, bn):
    def lk(bu_ref, bi_ref, bn_ref, o_ref):
        z = jnp.sum(bu_ref[...] * (bi_ref[...] - bn_ref[...]), axis=1)
        z = z * (1.0 / 16.0)
        sp = jnp.maximum(-z, 0.0) + jnp.log1p(jnp.exp(-jnp.abs(z)))
        o_ref[...] = jnp.mean(sp).reshape(1, 1)

    return pl.pallas_call(
        lk, out_shape=jax.ShapeDtypeStruct((1, 1), jnp.float32))(bu, bi, bn)


def kernel(user_embeddings, item_embeddings, edge_weight, edge_index,
           user_ids, item_ids, neg_item_ids):
    x = jnp.concatenate([user_embeddings, item_embeddings], axis=0)
    pad = E_PAD - E
    src = jnp.concatenate([edge_index[0], jnp.zeros((pad,), jnp.int32)])
    dst = jnp.concatenate([edge_index[1], jnp.zeros((pad,), jnp.int32)])
    padm = NCHUNK_META * CHUNK - E
    srcm = jnp.concatenate([edge_index[0], jnp.zeros((padm,), jnp.int32)])
    dstm = jnp.concatenate([edge_index[1], jnp.zeros((padm,), jnp.int32)])
    w = jnp.concatenate([edge_weight, jnp.zeros((padm,), jnp.float32)])
    w3 = w.reshape(NCHUNK_META, CHUNK)
    meta3 = jnp.concatenate([srcm.reshape(NCHUNK_META, CJ, CL),
                             dstm.reshape(NCHUNK_META, CJ, CL)], axis=1)

    h = x
    agg = x.reshape(N_PAD * D // 128, 128)
    for _ in range(3):
        parts = _layer_call(h, meta3, w3)
        h2, agg = _combine_call(parts.reshape(NC, N_PAD * D // 128, 128), agg)
        h = h2.reshape(N_PAD, D)

    uid2 = user_ids.reshape(NW, 128)
    iid2 = (item_ids + N_USER).reshape(NW, 128)
    nid2 = (neg_item_ids + N_USER).reshape(NW, 128)
    bu, bi, bn = _batch_gather_call(agg.reshape(N_PAD, D), uid2, iid2, nid2)
    loss2 = _loss_call(bu.reshape(4096, D), bi.reshape(4096, D),
                       bn.reshape(4096, D))
    return loss2[0, 0]
